# Initial kernel scaffold; baseline (speedup 1.0000x reference)
import jax
import jax.numpy as jnp
from jax.experimental import pallas as pl


def kernel(x0, v, beta, times_list, node_pairs):
    raise NotImplementedError("write your pallas kernel here")


# speedup vs baseline: 2.6903x; 2.6903x over previous
import functools

import jax
import jax.numpy as jnp
from jax import lax
from jax.experimental import pallas as pl
from jax.experimental.pallas import tpu as pltpu
from jax.experimental.pallas import tpu_sc as plsc

_W = 32
_PB = 256


def _sc_gather(table, idx_i, idx_j, ppad, f):
    mesh = plsc.VectorSubcoreMesh(core_axis_name="c", subcore_axis_name="s")
    n_workers = mesh.num_cores * mesh.num_subcores
    bpw = ppad // n_workers

    out_t = (
        jax.ShapeDtypeStruct((ppad, f), jnp.float32),
        jax.ShapeDtypeStruct((ppad, f), jnp.float32),
    )

    @functools.partial(
        pl.kernel, out_type=out_t, mesh=mesh,
        scratch_types=[
            pltpu.VMEM((bpw,), jnp.int32),
            pltpu.VMEM((bpw,), jnp.int32),
            pltpu.VMEM((_W, f), jnp.float32),
            pltpu.VMEM((_W, f), jnp.float32),
            pltpu.SemaphoreType.DMA,
            pltpu.SemaphoreType.DMA,
        ])
    def gather_kernel(table_hbm, ii_hbm, ij_hbm, gi_hbm, gj_hbm,
                      ii_v, ij_v, bufi, bufj, semi, semj):
        wid = lax.axis_index("s") * mesh.num_cores + lax.axis_index("c")
        base = wid * bpw
        pltpu.sync_copy(ii_hbm.at[pl.ds(base, bpw)], ii_v)
        pltpu.sync_copy(ij_hbm.at[pl.ds(base, bpw)], ij_v)

        @pl.loop(0, bpw, step=_W)
        def _(c):
            ci = pltpu.async_copy(
                table_hbm.at[ii_v.at[pl.ds(c, _W)]], bufi, semi)
            cj = pltpu.async_copy(
                table_hbm.at[ij_v.at[pl.ds(c, _W)]], bufj, semj)
            ci.wait()
            cj.wait()
            pltpu.sync_copy(bufi, gi_hbm.at[pl.ds(base + c, _W)])
            pltpu.sync_copy(bufj, gj_hbm.at[pl.ds(base + c, _W)])

    return gather_kernel(table, idx_i, idx_j)


def _tc_body(nbins, d, t_len, times_ref, gi_ref, gj_ref, out_ref, abc_ref):
    gi = gi_ref[...]
    gj = gj_ref[...]
    beta_col = d + nbins * d
    lane = lax.broadcasted_iota(jnp.int32, gi.shape, 1)
    x = jnp.where(lane == beta_col, gi + gj, gi - gj)
    xt = jnp.transpose(x)

    inv_w = jnp.float32(1.0 / nbins)
    acc = xt[0:d, :]
    for m in range(nbins):
        dvm = xt[d + d * m:d + d * (m + 1), :]
        abc_ref[m:m + 1, :] = jnp.sum(acc * acc, axis=0, keepdims=True)
        abc_ref[nbins + m:nbins + m + 1, :] = jnp.sum(
            acc * dvm, axis=0, keepdims=True)
        abc_ref[2 * nbins + m:2 * nbins + m + 1, :] = jnp.sum(
            dvm * dvm, axis=0, keepdims=True)
        acc = acc + dvm * inv_w

    t = times_ref[...]
    mt = jnp.clip(jnp.floor(t * nbins), 0.0, nbins - 1.0)
    r = t - mt * inv_w
    lane2 = lax.broadcasted_iota(jnp.int32, (t_len, 3 * nbins), 1)
    binl = (lane2 % nbins).astype(jnp.float32)
    coef = jnp.where(lane2 < nbins, jnp.float32(1.0),
                     jnp.where(lane2 < 2 * nbins, 2.0 * r, r * r))
    sel = jnp.where(binl == mt, coef, jnp.float32(0.0))

    norm2 = lax.dot_general(
        sel, abc_ref[...], (((1,), (0,)), ((), ())),
        preferred_element_type=jnp.float32,
        precision=lax.Precision.HIGHEST)
    bsum = xt[beta_col:beta_col + 1, :]
    out_ref[...] = jnp.exp(bsum - norm2)


def _tc_compute(times2d, gi, gj, nbins, d, f, ppad):
    t_len = times2d.shape[0]
    body = functools.partial(_tc_body, nbins, d, t_len)
    return pl.pallas_call(
        body,
        grid=(ppad // _PB,),
        in_specs=[
            pl.BlockSpec((t_len, 1), lambda p: (0, 0)),
            pl.BlockSpec((_PB, f), lambda p: (p, 0)),
            pl.BlockSpec((_PB, f), lambda p: (p, 0)),
        ],
        out_specs=pl.BlockSpec((t_len, _PB), lambda p: (0, p)),
        out_shape=jax.ShapeDtypeStruct((t_len, ppad), jnp.float32),
        scratch_shapes=[pltpu.VMEM((3 * nbins, _PB), jnp.float32)],
    )(times2d, gi, gj)


def kernel(x0, v, beta, times_list, node_pairs):
    n, d = x0.shape
    nbins = v.shape[0]
    t_len = times_list.shape[0]
    p = node_pairs.shape[1]

    vt = jnp.transpose(v, (1, 0, 2)).reshape(n, nbins * d)
    fraw = d + nbins * d + 1
    f = ((fraw + 127) // 128) * 128
    table = jnp.concatenate(
        [x0, vt, beta[:, None],
         jnp.zeros((n, f - fraw), jnp.float32)], axis=1)

    align = max(_W * 32, _PB)
    ppad = ((p + align - 1) // align) * align
    idx = jnp.pad(node_pairs, ((0, 0), (0, ppad - p)))
    gi, gj = _sc_gather(table, idx[0], idx[1], ppad, f)

    times2d = times_list[:, None]
    out = _tc_compute(times2d, gi, gj, nbins, d, f, ppad)
    return out[:, :p]


# speedup vs baseline: 2.8949x; 1.0761x over previous
import functools

import jax
import jax.numpy as jnp
from jax import lax
from jax.experimental import pallas as pl
from jax.experimental.pallas import tpu as pltpu
from jax.experimental.pallas import tpu_sc as plsc

_W = 16
_PB = 256


def _sc_gather(table, idx_i, idx_j, ppad, f):
    mesh = plsc.VectorSubcoreMesh(core_axis_name="c", subcore_axis_name="s")
    n_workers = mesh.num_cores * mesh.num_subcores
    bpw = ppad // n_workers

    out_t = (
        jax.ShapeDtypeStruct((ppad, f), jnp.float32),
        jax.ShapeDtypeStruct((ppad, f), jnp.float32),
    )

    nch = bpw // _W

    @functools.partial(
        pl.kernel, out_type=out_t, mesh=mesh,
        scratch_types=[
            pltpu.VMEM((bpw,), jnp.int32),
            pltpu.VMEM((bpw,), jnp.int32),
            pltpu.VMEM((_W, f), jnp.float32),
            pltpu.VMEM((_W, f), jnp.float32),
            pltpu.VMEM((_W, f), jnp.float32),
            pltpu.VMEM((_W, f), jnp.float32),
            pltpu.SemaphoreType.DMA,
            pltpu.SemaphoreType.DMA,
            pltpu.SemaphoreType.DMA,
            pltpu.SemaphoreType.DMA,
        ])
    def gather_kernel(table_hbm, ii_hbm, ij_hbm, gi_hbm, gj_hbm,
                      ii_v, ij_v, bi0, bj0, bi1, bj1, si0, sj0, si1, sj1):
        wid = lax.axis_index("s") * mesh.num_cores + lax.axis_index("c")
        base = wid * bpw
        pltpu.sync_copy(ii_hbm.at[pl.ds(base, bpw)], ii_v)
        pltpu.sync_copy(ij_hbm.at[pl.ds(base, bpw)], ij_v)

        def fire(c, bi, bj, si, sj):
            pltpu.make_async_copy(
                table_hbm.at[ii_v.at[pl.ds(c * _W, _W)]], bi, si).start()
            pltpu.make_async_copy(
                table_hbm.at[ij_v.at[pl.ds(c * _W, _W)]], bj, sj).start()

        def drain(c, bi, bj, si, sj):
            pltpu.make_async_copy(
                table_hbm.at[ii_v.at[pl.ds(c * _W, _W)]], bi, si).wait()
            pltpu.make_async_copy(
                table_hbm.at[ij_v.at[pl.ds(c * _W, _W)]], bj, sj).wait()
            pltpu.sync_copy(bi, gi_hbm.at[pl.ds(base + c * _W, _W)])
            pltpu.sync_copy(bj, gj_hbm.at[pl.ds(base + c * _W, _W)])

        fire(0, bi0, bj0, si0, sj0)

        @pl.loop(0, nch, step=2)
        def _(c):
            fire(c + 1, bi1, bj1, si1, sj1)
            drain(c, bi0, bj0, si0, sj0)

            @pl.when(c + 2 < nch)
            def _():
                fire(c + 2, bi0, bj0, si0, sj0)

            drain(c + 1, bi1, bj1, si1, sj1)

    return gather_kernel(table, idx_i, idx_j)


def _tc_body(nbins, d, t_len, times_ref, gi_ref, gj_ref, out_ref, abc_ref):
    gi = gi_ref[...]
    gj = gj_ref[...]
    beta_col = d + nbins * d
    lane = lax.broadcasted_iota(jnp.int32, gi.shape, 1)
    x = jnp.where(lane == beta_col, gi + gj, gi - gj)
    xt = jnp.transpose(x)

    inv_w = jnp.float32(1.0 / nbins)
    acc = xt[0:d, :]
    for m in range(nbins):
        dvm = xt[d + d * m:d + d * (m + 1), :]
        abc_ref[m:m + 1, :] = jnp.sum(acc * acc, axis=0, keepdims=True)
        abc_ref[nbins + m:nbins + m + 1, :] = jnp.sum(
            acc * dvm, axis=0, keepdims=True)
        abc_ref[2 * nbins + m:2 * nbins + m + 1, :] = jnp.sum(
            dvm * dvm, axis=0, keepdims=True)
        acc = acc + dvm * inv_w

    t = times_ref[...]
    mt = jnp.clip(jnp.floor(t * nbins), 0.0, nbins - 1.0)
    r = t - mt * inv_w
    lane2 = lax.broadcasted_iota(jnp.int32, (t_len, 3 * nbins), 1)
    binl = (lane2 % nbins).astype(jnp.float32)
    coef = jnp.where(lane2 < nbins, jnp.float32(1.0),
                     jnp.where(lane2 < 2 * nbins, 2.0 * r, r * r))
    sel = jnp.where(binl == mt, coef, jnp.float32(0.0))

    norm2 = lax.dot_general(
        sel, abc_ref[...], (((1,), (0,)), ((), ())),
        preferred_element_type=jnp.float32,
        precision=lax.Precision.HIGHEST)
    bsum = xt[beta_col:beta_col + 1, :]
    out_ref[...] = jnp.exp(bsum - norm2)


def _tc_compute(times2d, gi, gj, nbins, d, f, ppad):
    t_len = times2d.shape[0]
    body = functools.partial(_tc_body, nbins, d, t_len)
    return pl.pallas_call(
        body,
        grid=(ppad // _PB,),
        in_specs=[
            pl.BlockSpec((t_len, 1), lambda p: (0, 0)),
            pl.BlockSpec((_PB, f), lambda p: (p, 0)),
            pl.BlockSpec((_PB, f), lambda p: (p, 0)),
        ],
        out_specs=pl.BlockSpec((t_len, _PB), lambda p: (0, p)),
        out_shape=jax.ShapeDtypeStruct((t_len, ppad), jnp.float32),
        scratch_shapes=[pltpu.VMEM((3 * nbins, _PB), jnp.float32)],
    )(times2d, gi, gj)


def kernel(x0, v, beta, times_list, node_pairs):
    n, d = x0.shape
    nbins = v.shape[0]
    t_len = times_list.shape[0]
    p = node_pairs.shape[1]

    vt = jnp.transpose(v, (1, 0, 2)).reshape(n, nbins * d)
    fraw = d + nbins * d + 1
    f = ((fraw + 127) // 128) * 128
    table = jnp.concatenate(
        [x0, vt, beta[:, None],
         jnp.zeros((n, f - fraw), jnp.float32)], axis=1)

    align = max(_W * 32, _PB)
    ppad = ((p + align - 1) // align) * align
    idx = jnp.pad(node_pairs, ((0, 0), (0, ppad - p)))
    gi, gj = _sc_gather(table, idx[0], idx[1], ppad, f)

    times2d = times_list[:, None]
    out = _tc_compute(times2d, gi, gj, nbins, d, f, ppad)
    return out[:, :p]
